# baseline (speedup 1.0000x reference)
import functools

import jax
import jax.numpy as jnp
from jax import lax
from jax.experimental import pallas as pl
from jax.experimental.pallas import tpu as pltpu
from jax.experimental.pallas import tpu_sc as plsc

_INFO = plsc.get_sparse_core_info()
_NS = _INFO.num_subcores
_NW = _NS


@functools.lru_cache(maxsize=None)
def _build(batch: int):
    assert batch % _NW == 0
    b_per_w = batch // _NW
    mesh = plsc.VectorSubcoreMesh(
        core_axis_name="c", subcore_axis_name="s", num_cores=1)

    @functools.partial(
        pl.kernel,
        mesh=mesh,
        out_type=jax.ShapeDtypeStruct((batch,), jnp.float32),
        scratch_types=[
            pltpu.VMEM((b_per_w,), jnp.int32),
            pltpu.VMEM((b_per_w,), jnp.float32),
            pltpu.SemaphoreType.DMA,
        ],
    )
    def gather_kernel(item_hbm, scores_hbm, out_hbm, idx_v, vals_v, sem_g):
        wid = lax.axis_index("s")
        base = wid * b_per_w
        pltpu.sync_copy(item_hbm.at[pl.ds(base, b_per_w)], idx_v)
        pltpu.async_copy(scores_hbm.at[idx_v], vals_v, sem_g).wait()
        pltpu.sync_copy(vals_v, out_hbm.at[pl.ds(base, b_per_w)])

    return gather_kernel


def kernel(user_ids, item_ids, scores):
    del user_ids
    return _build(item_ids.shape[0])(item_ids.astype(jnp.int32), scores)


# speedup vs baseline: 1.0110x; 1.0110x over previous
import functools

import jax
import jax.numpy as jnp
from jax import lax
from jax.experimental import pallas as pl
from jax.experimental.pallas import tpu as pltpu
from jax.experimental.pallas import tpu_sc as plsc

_INFO = plsc.get_sparse_core_info()
_NC = _INFO.num_cores
_NS = _INFO.num_subcores
_NW = 1 * _NS
_CHUNKS = 2


@functools.lru_cache(maxsize=None)
def _build(batch: int):
    assert batch % (_NW * _CHUNKS) == 0
    b_per_w = batch // _NW
    csz = b_per_w // _CHUNKS
    mesh = plsc.VectorSubcoreMesh(
        core_axis_name="c", subcore_axis_name="s", num_cores=1)

    @functools.partial(
        pl.kernel,
        mesh=mesh,
        out_type=jax.ShapeDtypeStruct((batch,), jnp.float32),
        scratch_types=[
            pltpu.VMEM((b_per_w,), jnp.int32),
            pltpu.VMEM((b_per_w,), jnp.float32),
            pltpu.SemaphoreType.DMA,
            pltpu.SemaphoreType.DMA,
            pltpu.SemaphoreType.DMA,
        ],
    )
    def gather_kernel(item_hbm, scores_hbm, out_hbm, idx_v, vals_v,
                      sem_i, sem_g, sem_o):
        wid = lax.axis_index("s")
        base = wid * b_per_w
        idx_copies = [
            pltpu.async_copy(
                item_hbm.at[pl.ds(base + j * csz, csz)],
                idx_v.at[pl.ds(j * csz, csz)],
                sem_i,
            )
            for j in range(_CHUNKS)
        ]
        gathers = []
        for j in range(_CHUNKS):
            idx_copies[j].wait()
            gathers.append(
                pltpu.async_copy(
                    scores_hbm.at[idx_v.at[pl.ds(j * csz, csz)]],
                    vals_v.at[pl.ds(j * csz, csz)],
                    sem_g,
                )
            )
        out_copies = []
        for j in range(_CHUNKS):
            gathers[j].wait()
            out_copies.append(
                pltpu.async_copy(
                    vals_v.at[pl.ds(j * csz, csz)],
                    out_hbm.at[pl.ds(base + j * csz, csz)],
                    sem_o,
                )
            )
        for c in out_copies:
            c.wait()

    return gather_kernel


def kernel(user_ids, item_ids, scores):
    del user_ids
    return _build(item_ids.shape[0])(item_ids.astype(jnp.int32), scores)
